# baseline (speedup 1.0000x reference)
import functools

import jax
import jax.numpy as jnp
import numpy as np
from jax import lax
from jax.experimental import pallas as pl
from jax.experimental.pallas import tpu as pltpu
from jax.experimental.pallas import tpu_sc as plsc

B, S, D, H, M, K, FF = 2, 512, 1024, 16, 65536, 16, 4096
DH = D // H
R = B * S
CHUNK = 128
NCHUNK = M // CHUNK
MB = 2048
NMB = M // MB
CPB = MB // CHUNK
f32 = jnp.float32
bf16 = jnp.bfloat16

_CP = pltpu.CompilerParams(vmem_limit_bytes=128 * 1024 * 1024)


def _dot_t(a, b):
    return lax.dot_general(a.astype(bf16), b.astype(bf16),
                           (((1,), (1,)), ((), ())),
                           preferred_element_type=f32)


def _dot(a, b):
    return lax.dot_general(a.astype(bf16), b.astype(bf16),
                           (((1,), (0,)), ((), ())),
                           preferred_element_type=f32)


def _ln(y, g, b):
    mu = jnp.mean(y, axis=1, keepdims=True)
    yc = y - mu
    var = jnp.mean(yc * yc, axis=1, keepdims=True)
    return yc * lax.rsqrt(var + 1e-5) * g + b


def _attend(q, k, v, scale):
    s = _dot_t(q * scale, k).astype(bf16)
    m = jnp.max(s, axis=1, keepdims=True)
    p = jnp.exp(s - m)
    ps = jnp.sum(p, axis=1, keepdims=True)
    o = lax.dot_general(p, v.astype(bf16), (((1,), (0,)), ((), ())),
                        preferred_element_type=f32)
    return o / ps.astype(f32)


def _sortable(v):
    i = lax.bitcast_convert_type(v, jnp.int32)
    return jnp.where(i < 0, i ^ np.int32(0x7FFFFFFF), i)


def _erf(x):
    t = 1.0 / (1.0 + 0.3275911 * jnp.abs(x))
    y = 1.0 - (((((1.061405429 * t - 1.453152027) * t) + 1.421413741) * t
                - 0.284496736) * t + 0.254829592) * t * jnp.exp(-x * x)
    return jnp.sign(x) * y


def _gelu_exact(x):
    return 0.5 * x * (1.0 + _erf(x * np.float32(1.0 / np.sqrt(2.0))))


def _full(a):
    return pl.BlockSpec(a.shape, lambda *_: (0,) * a.ndim)


def _proj_body(x_ref, w_ref, b_ref, o_ref):
    o_ref[...] = _dot_t(x_ref[...], w_ref[...]) + b_ref[...]


def _proj(xf, w, b, nblk=2):
    rows = xf.shape[0] // nblk
    return pl.pallas_call(
        _proj_body,
        grid=(nblk,),
        in_specs=[pl.BlockSpec((rows, xf.shape[1]), lambda i: (i, 0)),
                  _full(w), _full(b)],
        out_specs=pl.BlockSpec((rows, w.shape[0]), lambda i: (i, 0)),
        out_shape=jax.ShapeDtypeStruct((xf.shape[0], w.shape[0]), f32),
        compiler_params=_CP,
    )(xf, w, b)


def _sa_body(q_ref, k_ref, v_ref, o_ref):
    scale = np.float32(1.0 / np.sqrt(DH))
    for j in range(2):
        sl = slice(j * DH, (j + 1) * DH)
        o_ref[:, sl] = _attend(q_ref[:, sl], k_ref[:, sl], v_ref[:, sl],
                               scale)


def _self_attn(qkv):
    hp = H // 2
    return pl.pallas_call(
        _sa_body,
        grid=(B, hp),
        in_specs=[pl.BlockSpec((S, 2 * DH), lambda b, h: (b, h)),
                  pl.BlockSpec((S, 2 * DH), lambda b, h: (b, hp + h)),
                  pl.BlockSpec((S, 2 * DH), lambda b, h: (b, 2 * hp + h))],
        out_specs=pl.BlockSpec((S, 2 * DH), lambda b, h: (b, h)),
        out_shape=jax.ShapeDtypeStruct((R, D), f32),
        compiler_params=_CP,
    )(qkv, qkv, qkv)


def _oln_body(x_ref, o_ref, w_ref, b_ref, g_ref, gb_ref, y_ref):
    a = _dot_t(o_ref[...], w_ref[...]) + b_ref[...]
    y_ref[...] = _ln(x_ref[...] + a, g_ref[...], gb_ref[...])


def _out_ln(xf, o, w, b, g, gb):
    return pl.pallas_call(
        _oln_body,
        grid=(B,),
        in_specs=[pl.BlockSpec((S, D), lambda i: (i, 0)),
                  pl.BlockSpec((S, D), lambda i: (i, 0)),
                  _full(w), _full(b), _full(g), _full(gb)],
        out_specs=pl.BlockSpec((S, D), lambda i: (i, 0)),
        out_shape=jax.ShapeDtypeStruct((R, D), f32),
        compiler_params=_CP,
    )(xf, o, w, b, g, gb)


def _sims_body(q_ref, mk_ref, sims_ref, cmax_ref):
    mk = mk_ref[...]
    inv = lax.rsqrt(jnp.maximum(jnp.sum(mk * mk, axis=1, keepdims=True),
                                1e-24))
    mkn = mk * inv
    s = _dot_t(q_ref[...], mkn)
    for c in range(CPB):
        sims_ref[:, c] = s[:, c * CHUNK:(c + 1) * CHUNK].reshape(R // 8, 8,
                                                                 CHUNK)
    cms = [jnp.max(s[:, j * CHUNK:(j + 1) * CHUNK], axis=1, keepdims=True)
           for j in range(CPB)]
    cmax_ref[0] = jnp.concatenate(cms, axis=1)


def _sims_chunkmax(x1f, mem_k):
    return pl.pallas_call(
        _sims_body,
        grid=(NMB,),
        in_specs=[pl.BlockSpec((R, D), lambda m: (0, 0)),
                  pl.BlockSpec((MB, D), lambda m: (m, 0))],
        out_specs=[pl.BlockSpec((R // 8, CPB, 8, CHUNK),
                                lambda m: (0, m, 0, 0)),
                   pl.BlockSpec((1, R, CPB), lambda m: (m, 0, 0))],
        out_shape=[jax.ShapeDtypeStruct((R // 8, NCHUNK, 8, CHUNK), f32),
                   jax.ShapeDtypeStruct((NMB, R, CPB), f32)],
        compiler_params=_CP,
    )(x1f, mem_k)


_RB3 = 128


def _chunk_topk_body(cm_ref, out_ref):
    cm = cm_ref[...]
    iota = lax.broadcasted_iota(jnp.int32, (_RB3, NCHUNK), 1)
    key = (_sortable(cm) & np.int32(~0x1FF)) | iota
    sel = []
    for _ in range(K):
        m = jnp.max(key, axis=1, keepdims=True)
        sel.append(m & np.int32(0x1FF))
        key = jnp.where(key == m, jnp.int32(-2147483648), key)
    out_ref[...] = jnp.concatenate(sel, axis=1)


def _chunk_topk(cm_rows):
    return pl.pallas_call(
        _chunk_topk_body,
        grid=(R // _RB3,),
        in_specs=[pl.BlockSpec((_RB3, NCHUNK), lambda i: (i, 0))],
        out_specs=pl.BlockSpec((_RB3, K), lambda i: (i, 0)),
        out_shape=jax.ShapeDtypeStruct((R, K), jnp.int32),
        compiler_params=_CP,
    )(cm_rows)


def _sc_gather(table, idx, d_cols, rows_chunk):
    n_rows = idx.shape[0]
    dt = table.dtype
    info = plsc.get_sparse_core_info()
    nw = info.num_cores * info.num_subcores
    rpw = n_rows // nw
    nck = rpw // rows_chunk
    idx2d = idx.reshape(nw * nck, rows_chunk)
    mesh = plsc.VectorSubcoreMesh(core_axis_name="c", subcore_axis_name="s")

    @functools.partial(
        pl.kernel, mesh=mesh,
        out_type=jax.ShapeDtypeStruct((n_rows, d_cols), dt),
        scratch_types=[
            pltpu.VMEM((nck, rows_chunk), jnp.int32),
            pltpu.VMEM((rows_chunk, d_cols), dt),
            pltpu.VMEM((rows_chunk, d_cols), dt),
            pltpu.SemaphoreType.DMA,
            pltpu.SemaphoreType.DMA,
        ],
    )
    def k(table_hbm, idx_hbm, out_hbm, idx_v, buf0, buf1, sem0, sem1):
        wid = lax.axis_index("s") * info.num_cores + lax.axis_index("c")
        base = wid * rpw
        pltpu.sync_copy(idx_hbm.at[pl.ds(wid * nck, nck)], idx_v)
        bufs = (buf0, buf1)
        sems = (sem0, sem1)
        pend = [pltpu.async_copy(table_hbm.at[idx_v.at[0]], buf0, sem0)]
        for c in range(nck):
            pend[c].wait()
            if c + 1 < nck:
                pend.append(pltpu.async_copy(table_hbm.at[idx_v.at[c + 1]],
                                             bufs[(c + 1) % 2],
                                             sems[(c + 1) % 2]))
            pltpu.sync_copy(bufs[c % 2],
                            out_hbm.at[pl.ds(base + c * rows_chunk,
                                             rows_chunk)])
    return k(table, idx2d)


_RB5 = 64


def _final_topk_body(cands_ref, cid_ref, idx_ref):
    cid = cid_ref[...]
    iota128 = lax.broadcasted_iota(jnp.int32, (_RB5, CHUNK), 1)
    gparts = [cid[:, i:i + 1] * CHUNK + iota128 for i in range(K)]
    gidx = jnp.concatenate(gparts, axis=1)
    cands = cands_ref[...]
    key = (_sortable(cands) & np.int32(-65536)) | gidx
    out = []
    for _ in range(K):
        m = jnp.max(key, axis=1, keepdims=True)
        out.append(m & np.int32(0xFFFF))
        key = jnp.where(key == m, jnp.int32(-2147483648), key)
    idx_ref[...] = jnp.concatenate(out, axis=1)


def _final_topk(cands, cid):
    return pl.pallas_call(
        _final_topk_body,
        grid=(R // _RB5,),
        in_specs=[pl.BlockSpec((_RB5, K * CHUNK), lambda i: (i, 0)),
                  pl.BlockSpec((_RB5, K), lambda i: (i, 0))],
        out_specs=pl.BlockSpec((_RB5, K), lambda i: (i, 0)),
        out_shape=jax.ShapeDtypeStruct((R, K), jnp.int32),
        compiler_params=_CP,
    )(cands, cid)


def _kv_body(r_ref, kw_ref, vw_ref, vb_ref, kp_ref, vp_ref):
    r = r_ref[...]
    kp_ref[...] = _dot_t(r, kw_ref[...]).astype(bf16)
    vp_ref[...] = (_dot_t(r, vw_ref[...]) + vb_ref[...]).astype(bf16)


def _kv_proj(retr, kw, vw, vb):
    nblk = 8
    rows = (R * K) // nblk
    return pl.pallas_call(
        _kv_body,
        grid=(nblk,),
        in_specs=[pl.BlockSpec((rows, D), lambda i: (i, 0)),
                  _full(kw), _full(vw), _full(vb)],
        out_specs=[pl.BlockSpec((rows, D), lambda i: (i, 0)),
                   pl.BlockSpec((rows, D), lambda i: (i, 0))],
        out_shape=[jax.ShapeDtypeStruct((R * K, D), bf16),
                   jax.ShapeDtypeStruct((R * K, D), bf16)],
        compiler_params=_CP,
    )(retr, kw, vw, vb)


_QB = 512


def _ma_body(q_ref, kp_ref, vp_ref, o_ref):
    scale = np.float32(1.0 / np.sqrt(DH))
    for j in range(2):
        sl = slice(j * DH, (j + 1) * DH)
        o_ref[:, sl] = _attend(q_ref[:, sl], kp_ref[:, sl], vp_ref[:, sl],
                               scale)


def _ma_attn(qm, kp, vp):
    nq = S // _QB
    return pl.pallas_call(
        _ma_body,
        grid=(B, H // 2, nq),
        in_specs=[pl.BlockSpec((_QB, 2 * DH), lambda b, h, q: (b * nq + q, h)),
                  pl.BlockSpec((S * K, 2 * DH), lambda b, h, q: (b, h)),
                  pl.BlockSpec((S * K, 2 * DH), lambda b, h, q: (b, h))],
        out_specs=pl.BlockSpec((_QB, 2 * DH), lambda b, h, q: (b * nq + q, h)),
        out_shape=jax.ShapeDtypeStruct((R, D), f32),
        compiler_params=_CP,
    )(qm, kp, vp)


_NFF = 4
_FFB = FF // _NFF


def _tail_body(x1_ref, o_ref, ow_ref, ob_ref, mg_ref, mb_ref,
               w1_ref, b1_ref, w2_ref, b2_ref, fg_ref, fb_ref, out_ref,
               x2_s, acc_s):
    fb = pl.program_id(1)

    @pl.when(fb == 0)
    def _init():
        m = _dot_t(o_ref[...], ow_ref[...]) + ob_ref[...]
        x2_s[...] = _ln(x1_ref[...] + m, mg_ref[...], mb_ref[...])
        acc_s[...] = jnp.zeros_like(acc_s)

    h = _gelu_exact(_dot_t(x2_s[...], w1_ref[...]) + b1_ref[...])
    acc_s[...] += _dot_t(h, w2_ref[...])

    @pl.when(fb == _NFF - 1)
    def _fin():
        y = x2_s[...] + acc_s[...] + b2_ref[...]
        out_ref[...] = _ln(y, fg_ref[...], fb_ref[...])


def _tail(x1f, o, ow, ob, mg, mb, w1, b1, w2, b2, fg, fb):
    return pl.pallas_call(
        _tail_body,
        grid=(B, _NFF),
        in_specs=[pl.BlockSpec((S, D), lambda b, f: (b, 0)),
                  pl.BlockSpec((S, D), lambda b, f: (b, 0)),
                  _full(ow), _full(ob), _full(mg), _full(mb),
                  pl.BlockSpec((_FFB, D), lambda b, f: (f, 0)),
                  pl.BlockSpec((1, _FFB), lambda b, f: (0, f)),
                  pl.BlockSpec((D, _FFB), lambda b, f: (0, f)),
                  _full(b2), _full(fg), _full(fb)],
        out_specs=pl.BlockSpec((S, D), lambda b, f: (b, 0)),
        out_shape=jax.ShapeDtypeStruct((R, D), f32),
        scratch_shapes=[pltpu.VMEM((S, D), f32), pltpu.VMEM((S, D), f32)],
        compiler_params=_CP,
    )(x1f, o, ow, ob, mg, mb, w1, b1, w2, b2, fg, fb)


def kernel(x, sa_in_w, sa_in_b, sa_out_w, sa_out_b, an_g, an_b, mem_k, mem_v,
           ma_in_w, ma_in_b, ma_out_w, ma_out_b, mn_g, mn_b, w1, b1, w2, b2,
           fn_g, fn_b):
    row = lambda v: v.reshape(1, -1)
    xf = x.reshape(R, D)

    qkv = _proj(xf, sa_in_w, row(sa_in_b))
    o_sa = _self_attn(qkv)
    x1f = _out_ln(xf, o_sa, sa_out_w, row(sa_out_b), row(an_g), row(an_b))

    sims, cmax = _sims_chunkmax(x1f, mem_k)
    cm_rows = jnp.transpose(cmax, (1, 0, 2)).reshape(R, NCHUNK)
    cid = _chunk_topk(cm_rows)

    r = jnp.arange(R, dtype=jnp.int32)[:, None]
    flat = ((((r >> 3) * NCHUNK + cid) << 3) + (r & 7)).reshape(R * K)
    cands = _sc_gather(sims.reshape(R // 8 * NCHUNK * 8, CHUNK), flat,
                       CHUNK, rows_chunk=128)
    idx = _final_topk(cands.reshape(R, K * CHUNK), cid)

    retr = _sc_gather(mem_v, idx.reshape(R * K), D, rows_chunk=32)

    qm = _proj(x1f, ma_in_w[:D], row(ma_in_b[:D]))
    kp, vp = _kv_proj(retr, ma_in_w[D:2 * D], ma_in_w[2 * D:],
                      row(ma_in_b[2 * D:]))
    o = _ma_attn(qm, kp, vp)

    out = _tail(x1f, o, ma_out_w, row(ma_out_b), row(mn_g), row(mn_b),
                w1, row(b1), w2, row(b2), row(fn_g), row(fn_b))
    return out.reshape(B, S, D)


# speedup vs baseline: 1.0141x; 1.0141x over previous
import functools

import jax
import jax.numpy as jnp
import numpy as np
from jax import lax
from jax.experimental import pallas as pl
from jax.experimental.pallas import tpu as pltpu
from jax.experimental.pallas import tpu_sc as plsc

B, S, D, H, M, K, FF = 2, 512, 1024, 16, 65536, 16, 4096
DH = D // H
R = B * S
CHUNK = 128
NCHUNK = M // CHUNK
MB = 2048
NMB = M // MB
CPB = MB // CHUNK
f32 = jnp.float32
bf16 = jnp.bfloat16

_CP = pltpu.CompilerParams(vmem_limit_bytes=128 * 1024 * 1024)


def _dot_t(a, b):
    return lax.dot_general(a.astype(bf16), b.astype(bf16),
                           (((1,), (1,)), ((), ())),
                           preferred_element_type=f32)


def _dot(a, b):
    return lax.dot_general(a.astype(bf16), b.astype(bf16),
                           (((1,), (0,)), ((), ())),
                           preferred_element_type=f32)


def _ln(y, g, b):
    mu = jnp.mean(y, axis=1, keepdims=True)
    yc = y - mu
    var = jnp.mean(yc * yc, axis=1, keepdims=True)
    return yc * lax.rsqrt(var + 1e-5) * g + b


def _attend(q, k, v, scale):
    s = _dot_t(q * scale, k).astype(bf16)
    m = jnp.max(s, axis=1, keepdims=True)
    p = jnp.exp(s - m)
    ps = jnp.sum(p, axis=1, keepdims=True)
    o = lax.dot_general(p, v.astype(bf16), (((1,), (0,)), ((), ())),
                        preferred_element_type=f32)
    return o / ps.astype(f32)


def _sortable(v):
    i = lax.bitcast_convert_type(v, jnp.int32)
    return jnp.where(i < 0, i ^ np.int32(0x7FFFFFFF), i)


def _erf(x):
    t = 1.0 / (1.0 + 0.3275911 * jnp.abs(x))
    y = 1.0 - (((((1.061405429 * t - 1.453152027) * t) + 1.421413741) * t
                - 0.284496736) * t + 0.254829592) * t * jnp.exp(-x * x)
    return jnp.sign(x) * y


def _gelu_exact(x):
    return 0.5 * x * (1.0 + _erf(x * np.float32(1.0 / np.sqrt(2.0))))


def _full(a):
    return pl.BlockSpec(a.shape, lambda *_: (0,) * a.ndim)


def _proj_body(x_ref, w_ref, b_ref, o_ref):
    o_ref[...] = _dot_t(x_ref[...], w_ref[...]) + b_ref[...]


def _proj(xf, w, b, nblk=2):
    rows = xf.shape[0] // nblk
    return pl.pallas_call(
        _proj_body,
        grid=(nblk,),
        in_specs=[pl.BlockSpec((rows, xf.shape[1]), lambda i: (i, 0)),
                  _full(w), _full(b)],
        out_specs=pl.BlockSpec((rows, w.shape[0]), lambda i: (i, 0)),
        out_shape=jax.ShapeDtypeStruct((xf.shape[0], w.shape[0]), f32),
        compiler_params=_CP,
    )(xf, w, b)


def _sa_body(q_ref, k_ref, v_ref, o_ref):
    scale = np.float32(1.0 / np.sqrt(DH))
    for j in range(2):
        sl = slice(j * DH, (j + 1) * DH)
        o_ref[:, sl] = _attend(q_ref[:, sl], k_ref[:, sl], v_ref[:, sl],
                               scale)


def _self_attn(qkv):
    hp = H // 2
    return pl.pallas_call(
        _sa_body,
        grid=(B, hp),
        in_specs=[pl.BlockSpec((S, 2 * DH), lambda b, h: (b, h)),
                  pl.BlockSpec((S, 2 * DH), lambda b, h: (b, hp + h)),
                  pl.BlockSpec((S, 2 * DH), lambda b, h: (b, 2 * hp + h))],
        out_specs=pl.BlockSpec((S, 2 * DH), lambda b, h: (b, h)),
        out_shape=jax.ShapeDtypeStruct((R, D), f32),
        compiler_params=_CP,
    )(qkv, qkv, qkv)


def _oln_body(x_ref, o_ref, w_ref, b_ref, g_ref, gb_ref, y_ref):
    a = _dot_t(o_ref[...], w_ref[...]) + b_ref[...]
    y_ref[...] = _ln(x_ref[...] + a, g_ref[...], gb_ref[...])


def _out_ln(xf, o, w, b, g, gb):
    return pl.pallas_call(
        _oln_body,
        grid=(B,),
        in_specs=[pl.BlockSpec((S, D), lambda i: (i, 0)),
                  pl.BlockSpec((S, D), lambda i: (i, 0)),
                  _full(w), _full(b), _full(g), _full(gb)],
        out_specs=pl.BlockSpec((S, D), lambda i: (i, 0)),
        out_shape=jax.ShapeDtypeStruct((R, D), f32),
        compiler_params=_CP,
    )(xf, o, w, b, g, gb)


def _sims_body(q_ref, mk_ref, sims_ref, cmax_ref):
    mk = mk_ref[...]
    inv = lax.rsqrt(jnp.maximum(jnp.sum(mk * mk, axis=1, keepdims=True),
                                1e-24))
    mkn = mk * inv
    s = _dot_t(q_ref[...], mkn)
    for c in range(CPB):
        sims_ref[:, c] = s[:, c * CHUNK:(c + 1) * CHUNK].reshape(R // 8, 8,
                                                                 CHUNK)
    cms = [jnp.max(s[:, j * CHUNK:(j + 1) * CHUNK], axis=1, keepdims=True)
           for j in range(CPB)]
    cmax_ref[0] = jnp.concatenate(cms, axis=1)


def _sims_chunkmax(x1f, mem_k):
    return pl.pallas_call(
        _sims_body,
        grid=(NMB,),
        in_specs=[pl.BlockSpec((R, D), lambda m: (0, 0)),
                  pl.BlockSpec((MB, D), lambda m: (m, 0))],
        out_specs=[pl.BlockSpec((R // 8, CPB, 8, CHUNK),
                                lambda m: (0, m, 0, 0)),
                   pl.BlockSpec((1, R, CPB), lambda m: (m, 0, 0))],
        out_shape=[jax.ShapeDtypeStruct((R // 8, NCHUNK, 8, CHUNK), f32),
                   jax.ShapeDtypeStruct((NMB, R, CPB), f32)],
        compiler_params=_CP,
    )(x1f, mem_k)


_RB3 = 128


def _chunk_topk_body(cm_ref, out_ref):
    cm = jnp.concatenate([cm_ref[i] for i in range(NMB)], axis=1)
    iota = lax.broadcasted_iota(jnp.int32, (_RB3, NCHUNK), 1)
    key = (_sortable(cm) & np.int32(~0x1FF)) | iota
    sel = []
    for _ in range(K):
        m = jnp.max(key, axis=1, keepdims=True)
        sel.append(m & np.int32(0x1FF))
        key = jnp.where(key == m, jnp.int32(-2147483648), key)
    out_ref[...] = jnp.concatenate(sel, axis=1)


def _chunk_topk(cmax):
    return pl.pallas_call(
        _chunk_topk_body,
        grid=(R // _RB3,),
        in_specs=[pl.BlockSpec((NMB, _RB3, CPB), lambda i: (0, i, 0))],
        out_specs=pl.BlockSpec((_RB3, K), lambda i: (i, 0)),
        out_shape=jax.ShapeDtypeStruct((R, K), jnp.int32),
        compiler_params=_CP,
    )(cmax)


def _sc_gather(table, idx, d_cols, rows_chunk):
    n_rows = idx.shape[0]
    dt = table.dtype
    info = plsc.get_sparse_core_info()
    nw = info.num_cores * info.num_subcores
    rpw = n_rows // nw
    nck = rpw // rows_chunk
    idx2d = idx.reshape(nw * nck, rows_chunk)
    mesh = plsc.VectorSubcoreMesh(core_axis_name="c", subcore_axis_name="s")

    @functools.partial(
        pl.kernel, mesh=mesh,
        out_type=jax.ShapeDtypeStruct((n_rows, d_cols), dt),
        scratch_types=[
            pltpu.VMEM((nck, rows_chunk), jnp.int32),
            pltpu.VMEM((rows_chunk, d_cols), dt),
            pltpu.VMEM((rows_chunk, d_cols), dt),
            pltpu.SemaphoreType.DMA,
            pltpu.SemaphoreType.DMA,
        ],
    )
    def k(table_hbm, idx_hbm, out_hbm, idx_v, buf0, buf1, sem0, sem1):
        wid = lax.axis_index("s") * info.num_cores + lax.axis_index("c")
        base = wid * rpw
        pltpu.sync_copy(idx_hbm.at[pl.ds(wid * nck, nck)], idx_v)
        bufs = (buf0, buf1)
        sems = (sem0, sem1)
        pend = [pltpu.async_copy(table_hbm.at[idx_v.at[0]], buf0, sem0)]
        for c in range(nck):
            pend[c].wait()
            if c + 1 < nck:
                pend.append(pltpu.async_copy(table_hbm.at[idx_v.at[c + 1]],
                                             bufs[(c + 1) % 2],
                                             sems[(c + 1) % 2]))
            pltpu.sync_copy(bufs[c % 2],
                            out_hbm.at[pl.ds(base + c * rows_chunk,
                                             rows_chunk)])
    return k(table, idx2d)


_RB5 = 64


def _final_topk_body(cands_ref, cid_ref, idx_ref):
    cid = cid_ref[...]
    iota128 = lax.broadcasted_iota(jnp.int32, (_RB5, CHUNK), 1)
    gparts = [cid[:, i:i + 1] * CHUNK + iota128 for i in range(K)]
    gidx = jnp.concatenate(gparts, axis=1)
    cands = cands_ref[...]
    key = (_sortable(cands) & np.int32(-65536)) | gidx
    out = []
    for _ in range(K):
        m = jnp.max(key, axis=1, keepdims=True)
        out.append(m & np.int32(0xFFFF))
        key = jnp.where(key == m, jnp.int32(-2147483648), key)
    idx_ref[...] = jnp.concatenate(out, axis=1)


def _final_topk(cands, cid):
    return pl.pallas_call(
        _final_topk_body,
        grid=(R // _RB5,),
        in_specs=[pl.BlockSpec((_RB5, K * CHUNK), lambda i: (i, 0)),
                  pl.BlockSpec((_RB5, K), lambda i: (i, 0))],
        out_specs=pl.BlockSpec((_RB5, K), lambda i: (i, 0)),
        out_shape=jax.ShapeDtypeStruct((R, K), jnp.int32),
        compiler_params=_CP,
    )(cands, cid)


def _kv_body(r_ref, kw_ref, vw_ref, vb_ref, kp_ref, vp_ref):
    r = r_ref[...]
    kp_ref[...] = _dot_t(r, kw_ref[...]).astype(bf16)
    vp_ref[...] = (_dot_t(r, vw_ref[...]) + vb_ref[...]).astype(bf16)


def _kv_proj(retr, kw, vw, vb):
    nblk = 8
    rows = (R * K) // nblk
    return pl.pallas_call(
        _kv_body,
        grid=(nblk,),
        in_specs=[pl.BlockSpec((rows, D), lambda i: (i, 0)),
                  _full(kw), _full(vw), _full(vb)],
        out_specs=[pl.BlockSpec((rows, D), lambda i: (i, 0)),
                   pl.BlockSpec((rows, D), lambda i: (i, 0))],
        out_shape=[jax.ShapeDtypeStruct((R * K, D), bf16),
                   jax.ShapeDtypeStruct((R * K, D), bf16)],
        compiler_params=_CP,
    )(retr, kw, vw, vb)


_QB = 512


def _ma_body(q_ref, kp_ref, vp_ref, o_ref):
    scale = np.float32(1.0 / np.sqrt(DH))
    for j in range(2):
        sl = slice(j * DH, (j + 1) * DH)
        o_ref[:, sl] = _attend(q_ref[:, sl], kp_ref[:, sl], vp_ref[:, sl],
                               scale)


def _ma_attn(qm, kp, vp):
    nq = S // _QB
    return pl.pallas_call(
        _ma_body,
        grid=(B, H // 2, nq),
        in_specs=[pl.BlockSpec((_QB, 2 * DH), lambda b, h, q: (b * nq + q, h)),
                  pl.BlockSpec((S * K, 2 * DH), lambda b, h, q: (b, h)),
                  pl.BlockSpec((S * K, 2 * DH), lambda b, h, q: (b, h))],
        out_specs=pl.BlockSpec((_QB, 2 * DH), lambda b, h, q: (b * nq + q, h)),
        out_shape=jax.ShapeDtypeStruct((R, D), f32),
        compiler_params=_CP,
    )(qm, kp, vp)


_NFF = 4
_FFB = FF // _NFF


def _tail_body(x1_ref, o_ref, ow_ref, ob_ref, mg_ref, mb_ref,
               w1_ref, b1_ref, w2_ref, b2_ref, fg_ref, fb_ref, out_ref,
               x2_s, acc_s):
    fb = pl.program_id(1)

    @pl.when(fb == 0)
    def _init():
        m = _dot_t(o_ref[...], ow_ref[...]) + ob_ref[...]
        x2_s[...] = _ln(x1_ref[...] + m, mg_ref[...], mb_ref[...])
        acc_s[...] = jnp.zeros_like(acc_s)

    h = _gelu_exact(_dot_t(x2_s[...], w1_ref[...]) + b1_ref[...])
    acc_s[...] += _dot_t(h, w2_ref[...])

    @pl.when(fb == _NFF - 1)
    def _fin():
        y = x2_s[...] + acc_s[...] + b2_ref[...]
        out_ref[...] = _ln(y, fg_ref[...], fb_ref[...])


def _tail(x1f, o, ow, ob, mg, mb, w1, b1, w2, b2, fg, fb):
    return pl.pallas_call(
        _tail_body,
        grid=(B, _NFF),
        in_specs=[pl.BlockSpec((S, D), lambda b, f: (b, 0)),
                  pl.BlockSpec((S, D), lambda b, f: (b, 0)),
                  _full(ow), _full(ob), _full(mg), _full(mb),
                  pl.BlockSpec((_FFB, D), lambda b, f: (f, 0)),
                  pl.BlockSpec((1, _FFB), lambda b, f: (0, f)),
                  pl.BlockSpec((D, _FFB), lambda b, f: (0, f)),
                  _full(b2), _full(fg), _full(fb)],
        out_specs=pl.BlockSpec((S, D), lambda b, f: (b, 0)),
        out_shape=jax.ShapeDtypeStruct((R, D), f32),
        scratch_shapes=[pltpu.VMEM((S, D), f32), pltpu.VMEM((S, D), f32)],
        compiler_params=_CP,
    )(x1f, o, ow, ob, mg, mb, w1, b1, w2, b2, fg, fb)


def kernel(x, sa_in_w, sa_in_b, sa_out_w, sa_out_b, an_g, an_b, mem_k, mem_v,
           ma_in_w, ma_in_b, ma_out_w, ma_out_b, mn_g, mn_b, w1, b1, w2, b2,
           fn_g, fn_b):
    row = lambda v: v.reshape(1, -1)
    xf = x.reshape(R, D)

    qkv = _proj(xf, sa_in_w, row(sa_in_b))
    o_sa = _self_attn(qkv)
    x1f = _out_ln(xf, o_sa, sa_out_w, row(sa_out_b), row(an_g), row(an_b))

    sims, cmax = _sims_chunkmax(x1f, mem_k)
    cid = _chunk_topk(cmax)

    r = jnp.arange(R, dtype=jnp.int32)[:, None]
    flat = ((((r >> 3) * NCHUNK + cid) << 3) + (r & 7)).reshape(R * K)
    cands = _sc_gather(sims.reshape(R // 8 * NCHUNK * 8, CHUNK), flat,
                       CHUNK, rows_chunk=128)
    idx = _final_topk(cands.reshape(R, K * CHUNK), cid)

    retr = _sc_gather(mem_v, idx.reshape(R * K), D, rows_chunk=32)

    qm = _proj(x1f, ma_in_w[:D], row(ma_in_b[:D]))
    kp, vp = _kv_proj(retr, ma_in_w[D:2 * D], ma_in_w[2 * D:],
                      row(ma_in_b[2 * D:]))
    o = _ma_attn(qm, kp, vp)

    out = _tail(x1f, o, ma_out_w, row(ma_out_b), row(mn_g), row(mn_b),
                w1, row(b1), w2, row(b2), row(fn_g), row(fn_b))
    return out.reshape(B, S, D)
